# Initial kernel scaffold; baseline (speedup 1.0000x reference)
import jax
import jax.numpy as jnp
from jax.experimental import pallas as pl


def kernel(x, edge_index, W1, b1, W2, b2, Wout, bout):
    raise NotImplementedError("write your pallas kernel here")


# speedup vs baseline: 17.0558x; 17.0558x over previous
import functools

import jax
import jax.numpy as jnp
from jax import lax
from jax.experimental import pallas as pl
from jax.experimental.pallas import tpu as pltpu
from jax.experimental.pallas import tpu_sc as plsc

NC = 2
NS = 16
L = 16
B = 128
CB = 8
CH = CB * B
_SC_PARAMS = pltpu.CompilerParams(use_tc_tiling_on_sc=False)


def _cdiv(a, b):
    return (a + b - 1) // b


def _make_deg(N, EP):
    ept = EP // (NC * NS)
    nchunk = ept // CH
    DS = 8 * _cdiv(_cdiv(N + 8, NS), 8)
    DACC = NS * DS
    ZB = 2048
    mesh = plsc.VectorSubcoreMesh(
        core_axis_name="c", subcore_axis_name="s", num_cores=NC, num_subcores=NS)

    @functools.partial(
        pl.kernel,
        out_type=jax.ShapeDtypeStruct((NC, DACC), jnp.float32),
        mesh=mesh,
        compiler_params=_SC_PARAMS,
        scratch_types=[
            pltpu.VMEM_SHARED((DACC,), jnp.float32),
            pltpu.VMEM((ZB,), jnp.float32),
            pltpu.VMEM((CB, B), jnp.int32),
            pltpu.VMEM((B,), jnp.float32),
        ],
    )
    def deg_kernel(dst2, degp, dacc, zb, dstg, ones):
        c = lax.axis_index("c")
        t = lax.axis_index("s")

        def fill(i, _):
            zb[pl.ds(i * L, L)] = jnp.zeros((L,), jnp.float32)
            return 0
        lax.fori_loop(0, ZB // L, fill, 0)
        for v in range(B // L):
            ones[pl.ds(v * L, L)] = jnp.full((L,), 1.0, jnp.float32)

        zlo = t * DS
        nfull, rem = DS // ZB, DS % ZB
        for k in range(nfull):
            pltpu.sync_copy(zb, dacc.at[pl.ds(zlo + k * ZB, ZB)])
        if rem:
            pltpu.sync_copy(zb.at[pl.ds(0, rem)],
                            dacc.at[pl.ds(zlo + nfull * ZB, rem)])
        plsc.subcore_barrier()

        rows_per_tile = ept // B
        rbase0 = (c * NS + t) * rows_per_tile

        def chunk(ci, _):
            pltpu.sync_copy(dst2.at[pl.ds(rbase0 + ci * CB, CB)], dstg)
            for j in range(CB):
                pltpu.sync_copy(ones, dacc.at[dstg.at[j]], add=True)
            return 0
        lax.fori_loop(0, nchunk, chunk, 0)
        plsc.subcore_barrier()
        pltpu.sync_copy(dacc.at[pl.ds(t * DS, DS)],
                        degp.at[c, pl.ds(t * DS, DS)])

    return deg_kernel, DACC


def _make_prop(N, EP):
    ea_t = EP // NS
    eb_t = EP // (NS * NC)
    ncA, ncB = ea_t // CH, eb_t // CH
    FS = N // NS
    AccR = NS * _cdiv(N + 8, NS)
    ZS = AccR // NS
    ZB = 256
    mesh = plsc.VectorSubcoreMesh(
        core_axis_name="c", subcore_axis_name="s", num_cores=NC, num_subcores=NS)

    @functools.partial(
        pl.kernel,
        out_type=(jax.ShapeDtypeStruct((NC, N, L), jnp.float32),
                  jax.ShapeDtypeStruct((NC, N, L), jnp.float32)),
        mesh=mesh,
        compiler_params=_SC_PARAMS,
        scratch_types=[
            pltpu.VMEM_SHARED((AccR, L), jnp.float32),
            pltpu.VMEM((ZB, L), jnp.float32),
            pltpu.VMEM((CB, B), jnp.int32),
            pltpu.VMEM((CB, B), jnp.int32),
            pltpu.VMEM((B,), jnp.int32),
            pltpu.VMEM((B,), jnp.int32),
            pltpu.VMEM((B, L), jnp.float32),
            pltpu.VMEM((B, L), jnp.float32),
            pltpu.SemaphoreType.DMA,
            pltpu.SemaphoreType.DMA,
        ],
    )
    def prop_kernel(src2, dst2, yflat, s01, s2p,
                    acc, zb, sstg, dstg, ib0, ib1, rb0, rb1, sem0, sem1):
        c = lax.axis_index("c")
        t = lax.axis_index("s")
        ibs, rbs, sems = (ib0, ib1), (rb0, rb1), (sem0, sem1)

        def fill(i, _):
            zb[i, :] = jnp.zeros((L,), jnp.float32)
            return 0
        lax.fori_loop(0, ZB, fill, 0)

        def prep(j, p, off):
            for v in range(B // L):
                ibs[p][pl.ds(v * L, L)] = sstg[j, pl.ds(v * L, L)] + off

        def do_pass(nchunks, rows_base, off, out_ref):
            zlo = t * ZS
            nfull, rem = ZS // ZB, ZS % ZB
            for k in range(nfull):
                pltpu.sync_copy(zb, acc.at[pl.ds(zlo + k * ZB, ZB)])
            if rem:
                pltpu.sync_copy(zb.at[pl.ds(0, rem)],
                                acc.at[pl.ds(zlo + nfull * ZB, rem)])
            plsc.subcore_barrier()

            def chunk(ci, _):
                rb = rows_base + ci * CB
                pltpu.sync_copy(src2.at[pl.ds(rb, CB)], sstg)
                pltpu.sync_copy(dst2.at[pl.ds(rb, CB)], dstg)
                prep(0, 0, off)
                d = pltpu.async_copy(yflat.at[ibs[0]], rbs[0], sems[0])
                for j in range(CB):
                    p = j % 2
                    dn = None
                    if j + 1 < CB:
                        q = (j + 1) % 2
                        prep(j + 1, q, off)
                        dn = pltpu.async_copy(yflat.at[ibs[q]], rbs[q], sems[q])
                    d.wait()
                    pltpu.sync_copy(rbs[p], acc.at[dstg.at[j]], add=True)
                    d = dn
                return 0
            lax.fori_loop(0, nchunks, chunk, 0)
            plsc.subcore_barrier()
            flo = t * FS
            pltpu.sync_copy(acc.at[pl.ds(flo, FS)],
                            out_ref.at[c, pl.ds(flo, FS)])
            plsc.subcore_barrier()

        do_pass(ncA, t * (ea_t // B), c * N, s01)
        do_pass(ncB, c * (EP // (2 * B)) + t * (eb_t // B), 2 * N, s2p)

    return prop_kernel


def _t1_body(x_ref, w1_ref, da_ref, db_ref, y3_ref, dinv_ref):
    deg = da_ref[...] + db_ref[...] + 1.0
    dinv = lax.rsqrt(deg)
    xw = jnp.dot(x_ref[...], w1_ref[...], preferred_element_type=jnp.float32)
    y = xw * dinv
    for s in range(3):
        y3_ref[s] = y[:, s * L:(s + 1) * L]
    dinv_ref[...] = dinv


def _t2_body(s01_ref, s2p_ref, y3_ref, dinv_ref, b1_ref, w2_ref, y3o_ref):
    S = jnp.concatenate(
        [s01_ref[0], s01_ref[1], s2p_ref[0] + s2p_ref[1]], axis=1)
    y1 = jnp.concatenate([y3_ref[0], y3_ref[1], y3_ref[2]], axis=1)
    dinv = dinv_ref[...]
    h = jnp.maximum(dinv * (S + y1) + b1_ref[...], 0.0)
    y2 = jnp.dot(h, w2_ref[...], preferred_element_type=jnp.float32) * dinv
    for s in range(3):
        y3o_ref[s] = y2[:, s * L:(s + 1) * L]


def _t3_body(s01_ref, s2p_ref, y3_ref, dinv_ref, b2_ref, wo_ref, bo_ref,
             out_ref):
    S = jnp.concatenate(
        [s01_ref[0], s01_ref[1], s2p_ref[0] + s2p_ref[1]], axis=1)
    y2 = jnp.concatenate([y3_ref[0], y3_ref[1], y3_ref[2]], axis=1)
    dinv = dinv_ref[...]
    h = jnp.maximum(dinv * (S + y2) + b2_ref[...], 0.0)
    out_ref[...] = (jnp.dot(h, wo_ref[...], preferred_element_type=jnp.float32)
                    + bo_ref[...])


def _row_spec(R, width):
    return pl.BlockSpec((R, width), lambda i: (i, 0))


def _full_spec(shape):
    return pl.BlockSpec(shape, lambda i: tuple(0 for _ in shape))


def _slab_spec(R):
    return pl.BlockSpec((3, R, L), lambda i: (0, i, 0))


def _pair_spec(R):
    return pl.BlockSpec((2, R, L), lambda i: (0, i, 0))


def kernel(x, edge_index, W1, b1, W2, b2, Wout, bout):
    N, DIN = x.shape
    DH = W1.shape[1]
    DOUT = Wout.shape[1]
    E = edge_index.shape[1]
    assert DH == 3 * L and N % NS == 0

    EP = _cdiv(E, NC * NS * CH) * NC * NS * CH
    src = edge_index[0].astype(jnp.int32)
    dst = edge_index[1].astype(jnp.int32)
    pad = EP - E
    if pad:
        padi = jnp.arange(pad, dtype=jnp.int32)
        src = jnp.concatenate([src, padi % 64])
        dst = jnp.concatenate([dst, N + (padi % 8)])
    src2 = src.reshape(EP // B, B)
    dst2 = dst.reshape(EP // B, B)

    deg_kernel, DACC = _make_deg(N, EP)
    degp = deg_kernel(dst2)
    degA = degp[0, :N].reshape(N, 1)
    degB = degp[1, :N].reshape(N, 1)

    R = 2000
    grid = (N // R,)
    t1 = pl.pallas_call(
        _t1_body,
        grid=grid,
        in_specs=[_row_spec(R, DIN), _full_spec((DIN, DH)),
                  _row_spec(R, 1), _row_spec(R, 1)],
        out_specs=[_slab_spec(R), _row_spec(R, 1)],
        out_shape=[jax.ShapeDtypeStruct((3, N, L), jnp.float32),
                   jax.ShapeDtypeStruct((N, 1), jnp.float32)],
    )
    y3_1, dinv = t1(x, W1, degA, degB)

    prop = _make_prop(N, EP)
    s01_1, s2p_1 = prop(src2, dst2, y3_1.reshape(3 * N, L))

    t2 = pl.pallas_call(
        _t2_body,
        grid=grid,
        in_specs=[_pair_spec(R), _pair_spec(R), _slab_spec(R),
                  _row_spec(R, 1), _full_spec((1, DH)), _full_spec((DH, DH))],
        out_specs=_slab_spec(R),
        out_shape=jax.ShapeDtypeStruct((3, N, L), jnp.float32),
    )
    y3_2 = t2(s01_1, s2p_1, y3_1, dinv, b1.reshape(1, DH), W2)

    s01_2, s2p_2 = prop(src2, dst2, y3_2.reshape(3 * N, L))

    t3 = pl.pallas_call(
        _t3_body,
        grid=grid,
        in_specs=[_pair_spec(R), _pair_spec(R), _slab_spec(R),
                  _row_spec(R, 1), _full_spec((1, DH)), _full_spec((DH, DOUT)),
                  _full_spec((1, DOUT))],
        out_specs=_row_spec(R, DOUT),
        out_shape=jax.ShapeDtypeStruct((N, DOUT), jnp.float32),
    )
    return t3(s01_2, s2p_2, y3_2, dinv, b2.reshape(1, DH), Wout,
              bout.reshape(1, DOUT))


# speedup vs baseline: 22.2504x; 1.3046x over previous
import functools

import jax
import jax.numpy as jnp
from jax import lax
from jax.experimental import pallas as pl
from jax.experimental.pallas import tpu as pltpu
from jax.experimental.pallas import tpu_sc as plsc

NC = 2
NS = 16
L = 16
B = 128
CB = 16
CH = CB * B
ND = 4
_SC_PARAMS = pltpu.CompilerParams(use_tc_tiling_on_sc=False)


def _cdiv(a, b):
    return (a + b - 1) // b


def _make_deg(N, EP):
    ept = EP // (NC * NS)
    nchunk = ept // CH
    DS = 8 * _cdiv(_cdiv(N + 16, NS), 8)
    DACC = NS * DS
    ZB = 2048
    mesh = plsc.VectorSubcoreMesh(
        core_axis_name="c", subcore_axis_name="s", num_cores=NC, num_subcores=NS)

    @functools.partial(
        pl.kernel,
        out_type=jax.ShapeDtypeStruct((NC, DACC), jnp.float32),
        mesh=mesh,
        compiler_params=_SC_PARAMS,
        scratch_types=[
            pltpu.VMEM_SHARED((DACC,), jnp.float32),
            pltpu.VMEM((ZB,), jnp.float32),
            pltpu.VMEM((CB, B), jnp.int32),
            pltpu.VMEM((B,), jnp.float32),
        ],
    )
    def deg_kernel(dst2, degp, dacc, zb, dstg, ones):
        c = lax.axis_index("c")
        t = lax.axis_index("s")

        def fill(i, _):
            zb[pl.ds(i * L, L)] = jnp.zeros((L,), jnp.float32)
            return 0
        lax.fori_loop(0, ZB // L, fill, 0)
        for v in range(B // L):
            ones[pl.ds(v * L, L)] = jnp.full((L,), 1.0, jnp.float32)

        zlo = t * DS
        nfull, rem = DS // ZB, DS % ZB
        for k in range(nfull):
            pltpu.sync_copy(zb, dacc.at[pl.ds(zlo + k * ZB, ZB)])
        if rem:
            pltpu.sync_copy(zb.at[pl.ds(0, rem)],
                            dacc.at[pl.ds(zlo + nfull * ZB, rem)])
        plsc.subcore_barrier()

        rows_per_tile = ept // B
        rbase0 = (c * NS + t) * rows_per_tile

        def chunk(ci, _):
            pltpu.sync_copy(dst2.at[pl.ds(rbase0 + ci * CB, CB)], dstg)
            for j in range(CB):
                pltpu.sync_copy(ones, dacc.at[dstg.at[j]], add=True)
            return 0
        lax.fori_loop(0, nchunk, chunk, 0)
        plsc.subcore_barrier()
        pltpu.sync_copy(dacc.at[pl.ds(t * DS, DS)],
                        degp.at[c, pl.ds(t * DS, DS)])

    return deg_kernel, DACC


def _make_prop(N, EP):
    ea_t = EP // NS
    eb_t = EP // (NS * NC)
    ncA, ncB = ea_t // CH, eb_t // CH
    FS = N // NS
    AccR = NS * _cdiv(N + 16, NS)
    ZS = AccR // NS
    ZB = 256
    mesh = plsc.VectorSubcoreMesh(
        core_axis_name="c", subcore_axis_name="s", num_cores=NC, num_subcores=NS)

    @functools.partial(
        pl.kernel,
        out_type=(jax.ShapeDtypeStruct((NC, N, L), jnp.float32),
                  jax.ShapeDtypeStruct((NC, N, L), jnp.float32)),
        mesh=mesh,
        compiler_params=_SC_PARAMS,
        scratch_types=[
            pltpu.VMEM_SHARED((AccR, L), jnp.float32),
            pltpu.VMEM((ZB, L), jnp.float32),
            pltpu.VMEM((CB, B), jnp.int32),
            pltpu.VMEM((CB, B), jnp.int32),
        ] + [pltpu.VMEM((B,), jnp.int32) for _ in range(ND)]
          + [pltpu.VMEM((B, L), jnp.float32) for _ in range(ND)]
          + [pltpu.SemaphoreType.DMA for _ in range(ND)],
    )
    def prop_kernel(src2, dst2, yflat, s01, s2p,
                    acc, zb, sstg, dstg, *bufs):
        c = lax.axis_index("c")
        t = lax.axis_index("s")
        ibs, rbs, sems = bufs[:ND], bufs[ND:2 * ND], bufs[2 * ND:]

        def fill(i, _):
            zb[i, :] = jnp.zeros((L,), jnp.float32)
            return 0
        lax.fori_loop(0, ZB, fill, 0)

        def prep(j, p, off):
            for v in range(B // L):
                ibs[p][pl.ds(v * L, L)] = sstg[j, pl.ds(v * L, L)] + off

        def do_pass(nchunks, rows_base, off, out_ref):
            zlo = t * ZS
            nfull, rem = ZS // ZB, ZS % ZB
            for k in range(nfull):
                pltpu.sync_copy(zb, acc.at[pl.ds(zlo + k * ZB, ZB)])
            if rem:
                pltpu.sync_copy(zb.at[pl.ds(0, rem)],
                                acc.at[pl.ds(zlo + nfull * ZB, rem)])
            plsc.subcore_barrier()

            def chunk(ci, _):
                rb = rows_base + ci * CB
                pltpu.sync_copy(src2.at[pl.ds(rb, CB)], sstg)
                pltpu.sync_copy(dst2.at[pl.ds(rb, CB)], dstg)
                descs = {}
                for j in range(min(ND, CB)):
                    prep(j, j % ND, off)
                    descs[j] = pltpu.async_copy(
                        yflat.at[ibs[j % ND]], rbs[j % ND], sems[j % ND])
                for j in range(CB):
                    p = j % ND
                    descs.pop(j).wait()
                    pltpu.sync_copy(rbs[p], acc.at[dstg.at[j]], add=True)
                    if j + ND < CB:
                        prep(j + ND, p, off)
                        descs[j + ND] = pltpu.async_copy(
                            yflat.at[ibs[p]], rbs[p], sems[p])
                return 0
            lax.fori_loop(0, nchunks, chunk, 0)
            plsc.subcore_barrier()
            flo = t * FS
            pltpu.sync_copy(acc.at[pl.ds(flo, FS)],
                            out_ref.at[c, pl.ds(flo, FS)])
            plsc.subcore_barrier()

        do_pass(ncA, t * (ea_t // B), c * N, s01)
        do_pass(ncB, c * (EP // (2 * B)) + t * (eb_t // B), 2 * N, s2p)

    return prop_kernel


def _t1_body(x_ref, w1_ref, da_ref, db_ref, y3_ref, dinv_ref):
    deg = da_ref[...] + db_ref[...] + 1.0
    dinv = lax.rsqrt(deg)
    xw = jnp.dot(x_ref[...], w1_ref[...], preferred_element_type=jnp.float32)
    y = xw * dinv
    for s in range(3):
        y3_ref[s] = y[:, s * L:(s + 1) * L]
    dinv_ref[...] = dinv


def _t2_body(s01_ref, s2p_ref, y3_ref, dinv_ref, b1_ref, w2_ref, y3o_ref):
    S = jnp.concatenate(
        [s01_ref[0], s01_ref[1], s2p_ref[0] + s2p_ref[1]], axis=1)
    y1 = jnp.concatenate([y3_ref[0], y3_ref[1], y3_ref[2]], axis=1)
    dinv = dinv_ref[...]
    h = jnp.maximum(dinv * (S + y1) + b1_ref[...], 0.0)
    y2 = jnp.dot(h, w2_ref[...], preferred_element_type=jnp.float32) * dinv
    for s in range(3):
        y3o_ref[s] = y2[:, s * L:(s + 1) * L]


def _t3_body(s01_ref, s2p_ref, y3_ref, dinv_ref, b2_ref, wo_ref, bo_ref,
             out_ref):
    S = jnp.concatenate(
        [s01_ref[0], s01_ref[1], s2p_ref[0] + s2p_ref[1]], axis=1)
    y2 = jnp.concatenate([y3_ref[0], y3_ref[1], y3_ref[2]], axis=1)
    dinv = dinv_ref[...]
    h = jnp.maximum(dinv * (S + y2) + b2_ref[...], 0.0)
    out_ref[...] = (jnp.dot(h, wo_ref[...], preferred_element_type=jnp.float32)
                    + bo_ref[...])


def _row_spec(R, width):
    return pl.BlockSpec((R, width), lambda i: (i, 0))


def _full_spec(shape):
    return pl.BlockSpec(shape, lambda i: tuple(0 for _ in shape))


def _slab_spec(R):
    return pl.BlockSpec((3, R, L), lambda i: (0, i, 0))


def _pair_spec(R):
    return pl.BlockSpec((2, R, L), lambda i: (0, i, 0))


def kernel(x, edge_index, W1, b1, W2, b2, Wout, bout):
    N, DIN = x.shape
    DH = W1.shape[1]
    DOUT = Wout.shape[1]
    E = edge_index.shape[1]
    assert DH == 3 * L and N % NS == 0

    EP = _cdiv(E, NC * NS * CH) * NC * NS * CH
    src = edge_index[0].astype(jnp.int32)
    dst = edge_index[1].astype(jnp.int32)
    pad = EP - E
    if pad:
        padi = jnp.arange(pad, dtype=jnp.int32)
        src = jnp.concatenate([src, padi % 64])
        dst = jnp.concatenate([dst, N + (padi % 16)])
    src2 = src.reshape(EP // B, B)
    dst2 = dst.reshape(EP // B, B)

    deg_kernel, DACC = _make_deg(N, EP)
    degp = deg_kernel(dst2)
    degA = degp[0, :N].reshape(N, 1)
    degB = degp[1, :N].reshape(N, 1)

    R = 2000
    grid = (N // R,)
    t1 = pl.pallas_call(
        _t1_body,
        grid=grid,
        in_specs=[_row_spec(R, DIN), _full_spec((DIN, DH)),
                  _row_spec(R, 1), _row_spec(R, 1)],
        out_specs=[_slab_spec(R), _row_spec(R, 1)],
        out_shape=[jax.ShapeDtypeStruct((3, N, L), jnp.float32),
                   jax.ShapeDtypeStruct((N, 1), jnp.float32)],
    )
    y3_1, dinv = t1(x, W1, degA, degB)

    prop = _make_prop(N, EP)
    s01_1, s2p_1 = prop(src2, dst2, y3_1.reshape(3 * N, L))

    t2 = pl.pallas_call(
        _t2_body,
        grid=grid,
        in_specs=[_pair_spec(R), _pair_spec(R), _slab_spec(R),
                  _row_spec(R, 1), _full_spec((1, DH)), _full_spec((DH, DH))],
        out_specs=_slab_spec(R),
        out_shape=jax.ShapeDtypeStruct((3, N, L), jnp.float32),
    )
    y3_2 = t2(s01_1, s2p_1, y3_1, dinv, b1.reshape(1, DH), W2)

    s01_2, s2p_2 = prop(src2, dst2, y3_2.reshape(3 * N, L))

    t3 = pl.pallas_call(
        _t3_body,
        grid=grid,
        in_specs=[_pair_spec(R), _pair_spec(R), _slab_spec(R),
                  _row_spec(R, 1), _full_spec((1, DH)), _full_spec((DH, DOUT)),
                  _full_spec((1, DOUT))],
        out_specs=_row_spec(R, DOUT),
        out_shape=jax.ShapeDtypeStruct((N, DOUT), jnp.float32),
    )
    return t3(s01_2, s2p_2, y3_2, dinv, b2.reshape(1, DH), Wout,
              bout.reshape(1, DOUT))


# speedup vs baseline: 25.1543x; 1.1305x over previous
import functools

import jax
import jax.numpy as jnp
from jax import lax
from jax.experimental import pallas as pl
from jax.experimental.pallas import tpu as pltpu
from jax.experimental.pallas import tpu_sc as plsc

NC = 2
NS = 16
L = 16
B = 128
CB = 16
CH = CB * B
ND = 4
_SC_PARAMS = pltpu.CompilerParams(use_tc_tiling_on_sc=False)


def _cdiv(a, b):
    return (a + b - 1) // b


def _make_deg(N, EP):
    ept = EP // (NC * NS)
    nchunk = ept // CH
    DS = 8 * _cdiv(_cdiv(N + 16, NS), 8)
    DACC = NS * DS
    ZB = 2048
    mesh = plsc.VectorSubcoreMesh(
        core_axis_name="c", subcore_axis_name="s", num_cores=NC, num_subcores=NS)

    @functools.partial(
        pl.kernel,
        out_type=jax.ShapeDtypeStruct((NC, DACC), jnp.float32),
        mesh=mesh,
        compiler_params=_SC_PARAMS,
        scratch_types=[
            pltpu.VMEM_SHARED((DACC,), jnp.float32),
            pltpu.VMEM((ZB,), jnp.float32),
            pltpu.VMEM((CB, B), jnp.int32),
            pltpu.VMEM((B,), jnp.float32),
        ],
    )
    def deg_kernel(dst2, degp, dacc, zb, dstg, ones):
        c = lax.axis_index("c")
        t = lax.axis_index("s")

        def fill(i, _):
            zb[pl.ds(i * L, L)] = jnp.zeros((L,), jnp.float32)
            return 0
        lax.fori_loop(0, ZB // L, fill, 0)
        for v in range(B // L):
            ones[pl.ds(v * L, L)] = jnp.full((L,), 1.0, jnp.float32)

        zlo = t * DS
        nfull, rem = DS // ZB, DS % ZB
        for k in range(nfull):
            pltpu.sync_copy(zb, dacc.at[pl.ds(zlo + k * ZB, ZB)])
        if rem:
            pltpu.sync_copy(zb.at[pl.ds(0, rem)],
                            dacc.at[pl.ds(zlo + nfull * ZB, rem)])
        plsc.subcore_barrier()

        rows_per_tile = ept // B
        rbase0 = (c * NS + t) * rows_per_tile

        def chunk(ci, _):
            pltpu.sync_copy(dst2.at[pl.ds(rbase0 + ci * CB, CB)], dstg)
            for j in range(CB):
                pltpu.sync_copy(ones, dacc.at[dstg.at[j]], add=True)
            return 0
        lax.fori_loop(0, nchunk, chunk, 0)
        plsc.subcore_barrier()
        pltpu.sync_copy(dacc.at[pl.ds(t * DS, DS)],
                        degp.at[c, pl.ds(t * DS, DS)])

    return deg_kernel, DACC


def _make_prop(N, EP):
    ea_t = EP // NS
    eb_t = EP // (NS * NC)
    ncA, ncB = ea_t // CH, eb_t // CH
    FS = N // NS
    AccR = NS * _cdiv(N + 16, NS)
    ZS = AccR // NS
    ZB = 256
    mesh = plsc.VectorSubcoreMesh(
        core_axis_name="c", subcore_axis_name="s", num_cores=NC, num_subcores=NS)

    @functools.partial(
        pl.kernel,
        out_type=(jax.ShapeDtypeStruct((NC, N, L), jnp.float32),
                  jax.ShapeDtypeStruct((NC, N, L), jnp.float32)),
        mesh=mesh,
        compiler_params=_SC_PARAMS,
        scratch_types=[
            pltpu.VMEM_SHARED((AccR, L), jnp.float32),
            pltpu.VMEM((ZB, L), jnp.float32),
            pltpu.VMEM((CB, B), jnp.int32),
            pltpu.VMEM((CB, B), jnp.int32),
        ] + [pltpu.VMEM((B,), jnp.int32) for _ in range(ND)]
          + [pltpu.VMEM((B, L), jnp.float32) for _ in range(ND)]
          + [pltpu.SemaphoreType.DMA for _ in range(ND)],
    )
    def prop_kernel(src2, dst2, yflat, s01, s2p,
                    acc, zb, sstg, dstg, *bufs):
        c = lax.axis_index("c")
        t = lax.axis_index("s")
        ibs, rbs, sems = bufs[:ND], bufs[ND:2 * ND], bufs[2 * ND:]

        def fill(i, _):
            zb[i, :] = jnp.zeros((L,), jnp.float32)
            return 0
        lax.fori_loop(0, ZB, fill, 0)

        def prep(j, p, soff):
            for v in range(B // L):
                ibs[p][pl.ds(v * L, L)] = sstg[j, pl.ds(v * L, L)] * 3 + soff

        def do_pass(nchunks, rows_base, off, out_ref):
            zlo = t * ZS
            nfull, rem = ZS // ZB, ZS % ZB
            for k in range(nfull):
                pltpu.sync_copy(zb, acc.at[pl.ds(zlo + k * ZB, ZB)])
            if rem:
                pltpu.sync_copy(zb.at[pl.ds(0, rem)],
                                acc.at[pl.ds(zlo + nfull * ZB, rem)])
            plsc.subcore_barrier()

            def chunk(ci, _):
                rb = rows_base + ci * CB
                pltpu.sync_copy(src2.at[pl.ds(rb, CB)], sstg)
                pltpu.sync_copy(dst2.at[pl.ds(rb, CB)], dstg)
                descs = {}
                for j in range(min(ND, CB)):
                    prep(j, j % ND, off)
                    descs[j] = pltpu.async_copy(
                        yflat.at[ibs[j % ND]], rbs[j % ND], sems[j % ND])
                for j in range(CB):
                    p = j % ND
                    descs.pop(j).wait()
                    pltpu.sync_copy(rbs[p], acc.at[dstg.at[j]], add=True)
                    if j + ND < CB:
                        prep(j + ND, p, off)
                        descs[j + ND] = pltpu.async_copy(
                            yflat.at[ibs[p]], rbs[p], sems[p])
                return 0
            lax.fori_loop(0, nchunks, chunk, 0)
            plsc.subcore_barrier()
            flo = t * FS
            pltpu.sync_copy(acc.at[pl.ds(flo, FS)],
                            out_ref.at[c, pl.ds(flo, FS)])
            plsc.subcore_barrier()

        do_pass(ncA, t * (ea_t // B), c, s01)
        do_pass(ncB, c * (EP // (2 * B)) + t * (eb_t // B), 2, s2p)

    return prop_kernel


def _t1_body(x_ref, w1_ref, da_ref, db_ref, y_ref, dinv_ref):
    deg = da_ref[...] + db_ref[...] + 1.0
    dinv = lax.rsqrt(deg)
    xw = jnp.dot(x_ref[...], w1_ref[...], preferred_element_type=jnp.float32)
    y_ref[...] = xw * dinv
    dinv_ref[...] = dinv


def _t2_body(s01_ref, s2p_ref, y_ref, dinv_ref, b1_ref, w2_ref, yo_ref):
    S = jnp.concatenate(
        [s01_ref[0], s01_ref[1], s2p_ref[0] + s2p_ref[1]], axis=1)
    dinv = dinv_ref[...]
    h = jnp.maximum(dinv * (S + y_ref[...]) + b1_ref[...], 0.0)
    yo_ref[...] = jnp.dot(h, w2_ref[...],
                          preferred_element_type=jnp.float32) * dinv


def _t3_body(s01_ref, s2p_ref, y_ref, dinv_ref, b2_ref, wo_ref, bo_ref,
             out_ref):
    S = jnp.concatenate(
        [s01_ref[0], s01_ref[1], s2p_ref[0] + s2p_ref[1]], axis=1)
    dinv = dinv_ref[...]
    h = jnp.maximum(dinv * (S + y_ref[...]) + b2_ref[...], 0.0)
    out_ref[...] = (jnp.dot(h, wo_ref[...], preferred_element_type=jnp.float32)
                    + bo_ref[...])


def _row_spec(R, width):
    return pl.BlockSpec((R, width), lambda i: (i, 0))


def _full_spec(shape):
    return pl.BlockSpec(shape, lambda i: tuple(0 for _ in shape))


def _slab_spec(R):
    return pl.BlockSpec((3, R, L), lambda i: (0, i, 0))


def _pair_spec(R):
    return pl.BlockSpec((2, R, L), lambda i: (0, i, 0))


def kernel(x, edge_index, W1, b1, W2, b2, Wout, bout):
    N, DIN = x.shape
    DH = W1.shape[1]
    DOUT = Wout.shape[1]
    E = edge_index.shape[1]
    assert DH == 3 * L and N % NS == 0

    EP = _cdiv(E, NC * NS * CH) * NC * NS * CH
    src = edge_index[0].astype(jnp.int32)
    dst = edge_index[1].astype(jnp.int32)
    pad = EP - E
    if pad:
        padi = jnp.arange(pad, dtype=jnp.int32)
        src = jnp.concatenate([src, padi % 64])
        dst = jnp.concatenate([dst, N + (padi % 16)])
    src2 = src.reshape(EP // B, B)
    dst2 = dst.reshape(EP // B, B)

    deg_kernel, DACC = _make_deg(N, EP)
    degp = deg_kernel(dst2)
    degA = degp[0, :N].reshape(N, 1)
    degB = degp[1, :N].reshape(N, 1)

    R = 2000
    grid = (N // R,)
    t1 = pl.pallas_call(
        _t1_body,
        grid=grid,
        in_specs=[_row_spec(R, DIN), _full_spec((DIN, DH)),
                  _row_spec(R, 1), _row_spec(R, 1)],
        out_specs=[_row_spec(R, DH), _row_spec(R, 1)],
        out_shape=[jax.ShapeDtypeStruct((N, DH), jnp.float32),
                   jax.ShapeDtypeStruct((N, 1), jnp.float32)],
    )
    y1, dinv = t1(x, W1, degA, degB)

    prop = _make_prop(N, EP)
    s01_1, s2p_1 = prop(src2, dst2, y1.reshape(3 * N, L))

    t2 = pl.pallas_call(
        _t2_body,
        grid=grid,
        in_specs=[_pair_spec(R), _pair_spec(R), _row_spec(R, DH),
                  _row_spec(R, 1), _full_spec((1, DH)), _full_spec((DH, DH))],
        out_specs=_row_spec(R, DH),
        out_shape=jax.ShapeDtypeStruct((N, DH), jnp.float32),
    )
    y2 = t2(s01_1, s2p_1, y1, dinv, b1.reshape(1, DH), W2)

    s01_2, s2p_2 = prop(src2, dst2, y2.reshape(3 * N, L))

    t3 = pl.pallas_call(
        _t3_body,
        grid=grid,
        in_specs=[_pair_spec(R), _pair_spec(R), _row_spec(R, DH),
                  _row_spec(R, 1), _full_spec((1, DH)), _full_spec((DH, DOUT)),
                  _full_spec((1, DOUT))],
        out_specs=_row_spec(R, DOUT),
        out_shape=jax.ShapeDtypeStruct((N, DOUT), jnp.float32),
    )
    return t3(s01_2, s2p_2, y2, dinv, b2.reshape(1, DH), Wout,
              bout.reshape(1, DOUT))


# speedup vs baseline: 26.2572x; 1.0438x over previous
import functools

import jax
import jax.numpy as jnp
from jax import lax
from jax.experimental import pallas as pl
from jax.experimental.pallas import tpu as pltpu
from jax.experimental.pallas import tpu_sc as plsc

NC = 2
NS = 16
L = 16
B = 128
CB = 16
CH = CB * B
ND = 4
NB = 8
_SC_PARAMS = pltpu.CompilerParams(use_tc_tiling_on_sc=False)


def _cdiv(a, b):
    return (a + b - 1) // b


def _make_deg(N, EP):
    ept = EP // (NC * NS)
    nchunk = ept // CH
    DS = 8 * _cdiv(_cdiv(N + 16, NS), 8)
    DACC = NS * DS
    ZB = 2048
    mesh = plsc.VectorSubcoreMesh(
        core_axis_name="c", subcore_axis_name="s", num_cores=NC, num_subcores=NS)

    @functools.partial(
        pl.kernel,
        out_type=jax.ShapeDtypeStruct((NC, DACC), jnp.float32),
        mesh=mesh,
        compiler_params=_SC_PARAMS,
        scratch_types=[
            pltpu.VMEM_SHARED((DACC,), jnp.float32),
            pltpu.VMEM((ZB,), jnp.float32),
            pltpu.VMEM((CB, B), jnp.int32),
            pltpu.VMEM((B,), jnp.float32),
        ],
    )
    def deg_kernel(dst2, degp, dacc, zb, dstg, ones):
        c = lax.axis_index("c")
        t = lax.axis_index("s")

        def fill(i, _):
            zb[pl.ds(i * L, L)] = jnp.zeros((L,), jnp.float32)
            return 0
        lax.fori_loop(0, ZB // L, fill, 0)
        for v in range(B // L):
            ones[pl.ds(v * L, L)] = jnp.full((L,), 1.0, jnp.float32)

        zlo = t * DS
        nfull, rem = DS // ZB, DS % ZB
        for k in range(nfull):
            pltpu.sync_copy(zb, dacc.at[pl.ds(zlo + k * ZB, ZB)])
        if rem:
            pltpu.sync_copy(zb.at[pl.ds(0, rem)],
                            dacc.at[pl.ds(zlo + nfull * ZB, rem)])
        plsc.subcore_barrier()

        rows_per_tile = ept // B
        rbase0 = (c * NS + t) * rows_per_tile

        def chunk(ci, _):
            pltpu.sync_copy(dst2.at[pl.ds(rbase0 + ci * CB, CB)], dstg)
            for j in range(CB):
                pltpu.sync_copy(ones, dacc.at[dstg.at[j]], add=True)
            return 0
        lax.fori_loop(0, nchunk, chunk, 0)
        plsc.subcore_barrier()
        pltpu.sync_copy(dacc.at[pl.ds(t * DS, DS)],
                        degp.at[c, pl.ds(t * DS, DS)])

    return deg_kernel, DACC


def _make_prop(N, EP):
    ea_t = EP // NS
    eb_t = EP // (NS * NC)
    ncA, ncB = ea_t // CH, eb_t // CH
    FS = N // NS
    AccR = NS * _cdiv(N + 16, NS)
    ZS = AccR // NS
    ZB = 256
    mesh = plsc.VectorSubcoreMesh(
        core_axis_name="c", subcore_axis_name="s", num_cores=NC, num_subcores=NS)

    @functools.partial(
        pl.kernel,
        out_type=(jax.ShapeDtypeStruct((NC, N, L), jnp.float32),
                  jax.ShapeDtypeStruct((NC, N, L), jnp.float32)),
        mesh=mesh,
        compiler_params=_SC_PARAMS,
        scratch_types=[
            pltpu.VMEM_SHARED((AccR, L), jnp.float32),
            pltpu.VMEM((ZB, L), jnp.float32),
            pltpu.VMEM((CB, B), jnp.int32),
            pltpu.VMEM((CB, B), jnp.int32),
        ] + [pltpu.VMEM((B,), jnp.int32) for _ in range(NB)]
          + [pltpu.VMEM((B, L), jnp.float32) for _ in range(NB)]
          + [pltpu.SemaphoreType.DMA for _ in range(2 * NB)],
    )
    def prop_kernel(src2, dst2, yflat, s01, s2p,
                    acc, zb, sstg, dstg, *bufs):
        c = lax.axis_index("c")
        t = lax.axis_index("s")
        ibs, rbs = bufs[:NB], bufs[NB:2 * NB]
        gsems, ssems = bufs[2 * NB:3 * NB], bufs[3 * NB:]

        def fill(i, _):
            zb[i, :] = jnp.zeros((L,), jnp.float32)
            return 0
        lax.fori_loop(0, ZB, fill, 0)

        def prep(j, p, soff):
            for v in range(B // L):
                ibs[p][pl.ds(v * L, L)] = sstg[j, pl.ds(v * L, L)] * 3 + soff

        def do_pass(nchunks, rows_base, off, out_ref):
            zlo = t * ZS
            nfull, rem = ZS // ZB, ZS % ZB
            for k in range(nfull):
                pltpu.sync_copy(zb, acc.at[pl.ds(zlo + k * ZB, ZB)])
            if rem:
                pltpu.sync_copy(zb.at[pl.ds(0, rem)],
                                acc.at[pl.ds(zlo + nfull * ZB, rem)])
            plsc.subcore_barrier()

            def chunk(ci, _):
                rb = rows_base + ci * CB
                pltpu.sync_copy(src2.at[pl.ds(rb, CB)], sstg)
                pltpu.sync_copy(dst2.at[pl.ds(rb, CB)], dstg)
                gd, sd = {}, {}
                for j in range(min(ND, CB)):
                    q = j % NB
                    prep(j, q, off)
                    gd[j] = pltpu.async_copy(yflat.at[ibs[q]], rbs[q], gsems[q])
                for j in range(CB):
                    p = j % NB
                    gd.pop(j).wait()
                    sd[j] = pltpu.async_copy(
                        rbs[p], acc.at[dstg.at[j]], ssems[p], add=True)
                    nj = j + ND
                    if nj < CB:
                        q = nj % NB
                        if nj - NB in sd:
                            sd.pop(nj - NB).wait()
                        prep(nj, q, off)
                        gd[nj] = pltpu.async_copy(
                            yflat.at[ibs[q]], rbs[q], gsems[q])
                for j in sorted(sd):
                    sd.pop(j).wait()
                return 0
            lax.fori_loop(0, nchunks, chunk, 0)
            plsc.subcore_barrier()
            flo = t * FS
            pltpu.sync_copy(acc.at[pl.ds(flo, FS)],
                            out_ref.at[c, pl.ds(flo, FS)])
            plsc.subcore_barrier()

        do_pass(ncA, t * (ea_t // B), c, s01)
        do_pass(ncB, c * (EP // (2 * B)) + t * (eb_t // B), 2, s2p)

    return prop_kernel


def _t1_body(x_ref, w1_ref, da_ref, db_ref, y_ref, dinv_ref):
    deg = da_ref[...] + db_ref[...] + 1.0
    dinv = lax.rsqrt(deg)
    xw = jnp.dot(x_ref[...], w1_ref[...], preferred_element_type=jnp.float32)
    y_ref[...] = xw * dinv
    dinv_ref[...] = dinv


def _t2_body(s01_ref, s2p_ref, y_ref, dinv_ref, b1_ref, w2_ref, yo_ref):
    S = jnp.concatenate(
        [s01_ref[0], s01_ref[1], s2p_ref[0] + s2p_ref[1]], axis=1)
    dinv = dinv_ref[...]
    h = jnp.maximum(dinv * (S + y_ref[...]) + b1_ref[...], 0.0)
    yo_ref[...] = jnp.dot(h, w2_ref[...],
                          preferred_element_type=jnp.float32) * dinv


def _t3_body(s01_ref, s2p_ref, y_ref, dinv_ref, b2_ref, wo_ref, bo_ref,
             out_ref):
    S = jnp.concatenate(
        [s01_ref[0], s01_ref[1], s2p_ref[0] + s2p_ref[1]], axis=1)
    dinv = dinv_ref[...]
    h = jnp.maximum(dinv * (S + y_ref[...]) + b2_ref[...], 0.0)
    out_ref[...] = (jnp.dot(h, wo_ref[...], preferred_element_type=jnp.float32)
                    + bo_ref[...])


def _row_spec(R, width):
    return pl.BlockSpec((R, width), lambda i: (i, 0))


def _full_spec(shape):
    return pl.BlockSpec(shape, lambda i: tuple(0 for _ in shape))


def _slab_spec(R):
    return pl.BlockSpec((3, R, L), lambda i: (0, i, 0))


def _pair_spec(R):
    return pl.BlockSpec((2, R, L), lambda i: (0, i, 0))


def kernel(x, edge_index, W1, b1, W2, b2, Wout, bout):
    N, DIN = x.shape
    DH = W1.shape[1]
    DOUT = Wout.shape[1]
    E = edge_index.shape[1]
    assert DH == 3 * L and N % NS == 0

    EP = _cdiv(E, NC * NS * CH) * NC * NS * CH
    src = edge_index[0].astype(jnp.int32)
    dst = edge_index[1].astype(jnp.int32)
    pad = EP - E
    if pad:
        padi = jnp.arange(pad, dtype=jnp.int32)
        src = jnp.concatenate([src, padi % 64])
        dst = jnp.concatenate([dst, N + (padi % 16)])
    src2 = src.reshape(EP // B, B)
    dst2 = dst.reshape(EP // B, B)

    deg_kernel, DACC = _make_deg(N, EP)
    degp = deg_kernel(dst2)
    degA = degp[0, :N].reshape(N, 1)
    degB = degp[1, :N].reshape(N, 1)

    R = 2000
    grid = (N // R,)
    t1 = pl.pallas_call(
        _t1_body,
        grid=grid,
        in_specs=[_row_spec(R, DIN), _full_spec((DIN, DH)),
                  _row_spec(R, 1), _row_spec(R, 1)],
        out_specs=[_row_spec(R, DH), _row_spec(R, 1)],
        out_shape=[jax.ShapeDtypeStruct((N, DH), jnp.float32),
                   jax.ShapeDtypeStruct((N, 1), jnp.float32)],
    )
    y1, dinv = t1(x, W1, degA, degB)

    prop = _make_prop(N, EP)
    s01_1, s2p_1 = prop(src2, dst2, y1.reshape(3 * N, L))

    t2 = pl.pallas_call(
        _t2_body,
        grid=grid,
        in_specs=[_pair_spec(R), _pair_spec(R), _row_spec(R, DH),
                  _row_spec(R, 1), _full_spec((1, DH)), _full_spec((DH, DH))],
        out_specs=_row_spec(R, DH),
        out_shape=jax.ShapeDtypeStruct((N, DH), jnp.float32),
    )
    y2 = t2(s01_1, s2p_1, y1, dinv, b1.reshape(1, DH), W2)

    s01_2, s2p_2 = prop(src2, dst2, y2.reshape(3 * N, L))

    t3 = pl.pallas_call(
        _t3_body,
        grid=grid,
        in_specs=[_pair_spec(R), _pair_spec(R), _row_spec(R, DH),
                  _row_spec(R, 1), _full_spec((1, DH)), _full_spec((DH, DOUT)),
                  _full_spec((1, DOUT))],
        out_specs=_row_spec(R, DOUT),
        out_shape=jax.ShapeDtypeStruct((N, DOUT), jnp.float32),
    )
    return t3(s01_2, s2p_2, y2, dinv, b2.reshape(1, DH), Wout,
              bout.reshape(1, DOUT))


# speedup vs baseline: 28.1213x; 1.0710x over previous
import functools

import jax
import jax.numpy as jnp
from jax import lax
from jax.experimental import pallas as pl
from jax.experimental.pallas import tpu as pltpu
from jax.experimental.pallas import tpu_sc as plsc

NC = 2
NS = 16
L = 16
B = 128
CB = 16
CH = CB * B
ND = 6
NB = 8
_SC_PARAMS = pltpu.CompilerParams(use_tc_tiling_on_sc=False)


def _cdiv(a, b):
    return (a + b - 1) // b


def _make_deg(N, EP):
    ept = EP // (NC * NS)
    nchunk = ept // CH
    DS = 8 * _cdiv(_cdiv(N + 16, NS), 8)
    DACC = NS * DS
    ZB = 2048
    mesh = plsc.VectorSubcoreMesh(
        core_axis_name="c", subcore_axis_name="s", num_cores=NC, num_subcores=NS)

    @functools.partial(
        pl.kernel,
        out_type=jax.ShapeDtypeStruct((NC, DACC), jnp.float32),
        mesh=mesh,
        compiler_params=_SC_PARAMS,
        scratch_types=[
            pltpu.VMEM_SHARED((DACC,), jnp.float32),
            pltpu.VMEM((ZB,), jnp.float32),
            pltpu.VMEM((CB, B), jnp.int32),
            pltpu.VMEM((B,), jnp.float32),
        ],
    )
    def deg_kernel(dst2, degp, dacc, zb, dstg, ones):
        c = lax.axis_index("c")
        t = lax.axis_index("s")

        def fill(i, _):
            zb[pl.ds(i * L, L)] = jnp.zeros((L,), jnp.float32)
            return 0
        lax.fori_loop(0, ZB // L, fill, 0)
        for v in range(B // L):
            ones[pl.ds(v * L, L)] = jnp.full((L,), 1.0, jnp.float32)

        zlo = t * DS
        nfull, rem = DS // ZB, DS % ZB
        for k in range(nfull):
            pltpu.sync_copy(zb, dacc.at[pl.ds(zlo + k * ZB, ZB)])
        if rem:
            pltpu.sync_copy(zb.at[pl.ds(0, rem)],
                            dacc.at[pl.ds(zlo + nfull * ZB, rem)])
        plsc.subcore_barrier()

        rows_per_tile = ept // B
        rbase0 = (c * NS + t) * rows_per_tile

        def chunk(ci, _):
            pltpu.sync_copy(dst2.at[pl.ds(rbase0 + ci * CB, CB)], dstg)
            for j in range(CB):
                pltpu.sync_copy(ones, dacc.at[dstg.at[j]], add=True)
            return 0
        lax.fori_loop(0, nchunk, chunk, 0)
        plsc.subcore_barrier()
        pltpu.sync_copy(dacc.at[pl.ds(t * DS, DS)],
                        degp.at[c, pl.ds(t * DS, DS)])

    return deg_kernel, DACC


def _make_prop(N, EP):
    ea_t = EP // NS
    eb_t = EP // (NS * NC)
    ncA, ncB = ea_t // CH, eb_t // CH
    FS = N // NS
    AccR = NS * _cdiv(N + 16, NS)
    ZS = AccR // NS
    ZB = 256
    mesh = plsc.VectorSubcoreMesh(
        core_axis_name="c", subcore_axis_name="s", num_cores=NC, num_subcores=NS)

    @functools.partial(
        pl.kernel,
        out_type=(jax.ShapeDtypeStruct((NC, N, L), jnp.float32),
                  jax.ShapeDtypeStruct((NC, N, L), jnp.float32)),
        mesh=mesh,
        compiler_params=_SC_PARAMS,
        scratch_types=[
            pltpu.VMEM_SHARED((AccR, L), jnp.float32),
            pltpu.VMEM((ZB, L), jnp.float32),
            pltpu.VMEM((CB, B), jnp.int32),
            pltpu.VMEM((CB, B), jnp.int32),
        ] + [pltpu.VMEM((B,), jnp.int32) for _ in range(NB)]
          + [pltpu.VMEM((B, L), jnp.float32) for _ in range(NB)]
          + [pltpu.SemaphoreType.DMA for _ in range(2 * NB)],
    )
    def prop_kernel(src2, dst2, yflat, s01, s2p,
                    acc, zb, sstg, dstg, *bufs):
        c = lax.axis_index("c")
        t = lax.axis_index("s")
        ibs, rbs = bufs[:NB], bufs[NB:2 * NB]
        gsems, ssems = bufs[2 * NB:3 * NB], bufs[3 * NB:]

        def fill(i, _):
            zb[i, :] = jnp.zeros((L,), jnp.float32)
            return 0
        lax.fori_loop(0, ZB, fill, 0)

        def prep(j, p, soff):
            for v in range(B // L):
                ibs[p][pl.ds(v * L, L)] = sstg[j, pl.ds(v * L, L)] * 3 + soff

        def do_pass(nchunks, rows_base, off, out_ref):
            zlo = t * ZS
            nfull, rem = ZS // ZB, ZS % ZB
            for k in range(nfull):
                pltpu.sync_copy(zb, acc.at[pl.ds(zlo + k * ZB, ZB)])
            if rem:
                pltpu.sync_copy(zb.at[pl.ds(0, rem)],
                                acc.at[pl.ds(zlo + nfull * ZB, rem)])
            plsc.subcore_barrier()

            def chunk(ci, _):
                rb = rows_base + ci * CB
                pltpu.sync_copy(src2.at[pl.ds(rb, CB)], sstg)
                pltpu.sync_copy(dst2.at[pl.ds(rb, CB)], dstg)
                gd, sd = {}, {}
                for j in range(min(ND, CB)):
                    q = j % NB
                    prep(j, q, off)
                    gd[j] = pltpu.async_copy(yflat.at[ibs[q]], rbs[q], gsems[q])
                for j in range(CB):
                    p = j % NB
                    gd.pop(j).wait()
                    sd[j] = pltpu.async_copy(
                        rbs[p], acc.at[dstg.at[j]], ssems[p], add=True)
                    nj = j + ND
                    if nj < CB:
                        q = nj % NB
                        if nj - NB in sd:
                            sd.pop(nj - NB).wait()
                        prep(nj, q, off)
                        gd[nj] = pltpu.async_copy(
                            yflat.at[ibs[q]], rbs[q], gsems[q])
                for j in sorted(sd):
                    sd.pop(j).wait()
                return 0
            lax.fori_loop(0, nchunks, chunk, 0)
            plsc.subcore_barrier()
            flo = t * FS
            pltpu.sync_copy(acc.at[pl.ds(flo, FS)],
                            out_ref.at[c, pl.ds(flo, FS)])
            plsc.subcore_barrier()

        do_pass(ncA, t * (ea_t // B), c, s01)
        do_pass(ncB, c * (EP // (2 * B)) + t * (eb_t // B), 2, s2p)

    return prop_kernel


def _t1_body(x_ref, w1_ref, da_ref, db_ref, y_ref, dinv_ref):
    deg = da_ref[...] + db_ref[...] + 1.0
    dinv = lax.rsqrt(deg)
    xw = jnp.dot(x_ref[...], w1_ref[...], preferred_element_type=jnp.float32)
    y_ref[...] = xw * dinv
    dinv_ref[...] = dinv


def _t2_body(s01_ref, s2p_ref, y_ref, dinv_ref, b1_ref, w2_ref, yo_ref):
    S = jnp.concatenate(
        [s01_ref[0], s01_ref[1], s2p_ref[0] + s2p_ref[1]], axis=1)
    dinv = dinv_ref[...]
    h = jnp.maximum(dinv * (S + y_ref[...]) + b1_ref[...], 0.0)
    yo_ref[...] = jnp.dot(h, w2_ref[...],
                          preferred_element_type=jnp.float32) * dinv


def _t3_body(s01_ref, s2p_ref, y_ref, dinv_ref, b2_ref, wo_ref, bo_ref,
             out_ref):
    S = jnp.concatenate(
        [s01_ref[0], s01_ref[1], s2p_ref[0] + s2p_ref[1]], axis=1)
    dinv = dinv_ref[...]
    h = jnp.maximum(dinv * (S + y_ref[...]) + b2_ref[...], 0.0)
    out_ref[...] = (jnp.dot(h, wo_ref[...], preferred_element_type=jnp.float32)
                    + bo_ref[...])


def _row_spec(R, width):
    return pl.BlockSpec((R, width), lambda i: (i, 0))


def _full_spec(shape):
    return pl.BlockSpec(shape, lambda i: tuple(0 for _ in shape))


def _slab_spec(R):
    return pl.BlockSpec((3, R, L), lambda i: (0, i, 0))


def _pair_spec(R):
    return pl.BlockSpec((2, R, L), lambda i: (0, i, 0))


def kernel(x, edge_index, W1, b1, W2, b2, Wout, bout):
    N, DIN = x.shape
    DH = W1.shape[1]
    DOUT = Wout.shape[1]
    E = edge_index.shape[1]
    assert DH == 3 * L and N % NS == 0

    EP = _cdiv(E, NC * NS * CH) * NC * NS * CH
    src = edge_index[0].astype(jnp.int32)
    dst = edge_index[1].astype(jnp.int32)
    pad = EP - E
    if pad:
        padi = jnp.arange(pad, dtype=jnp.int32)
        src = jnp.concatenate([src, padi % 64])
        dst = jnp.concatenate([dst, N + (padi % 16)])
    src2 = src.reshape(EP // B, B)
    dst2 = dst.reshape(EP // B, B)

    deg_kernel, DACC = _make_deg(N, EP)
    degp = deg_kernel(dst2)
    degA = degp[0, :N].reshape(N, 1)
    degB = degp[1, :N].reshape(N, 1)

    R = 2000
    grid = (N // R,)
    t1 = pl.pallas_call(
        _t1_body,
        grid=grid,
        in_specs=[_row_spec(R, DIN), _full_spec((DIN, DH)),
                  _row_spec(R, 1), _row_spec(R, 1)],
        out_specs=[_row_spec(R, DH), _row_spec(R, 1)],
        out_shape=[jax.ShapeDtypeStruct((N, DH), jnp.float32),
                   jax.ShapeDtypeStruct((N, 1), jnp.float32)],
    )
    y1, dinv = t1(x, W1, degA, degB)

    prop = _make_prop(N, EP)
    s01_1, s2p_1 = prop(src2, dst2, y1.reshape(3 * N, L))

    t2 = pl.pallas_call(
        _t2_body,
        grid=grid,
        in_specs=[_pair_spec(R), _pair_spec(R), _row_spec(R, DH),
                  _row_spec(R, 1), _full_spec((1, DH)), _full_spec((DH, DH))],
        out_specs=_row_spec(R, DH),
        out_shape=jax.ShapeDtypeStruct((N, DH), jnp.float32),
    )
    y2 = t2(s01_1, s2p_1, y1, dinv, b1.reshape(1, DH), W2)

    s01_2, s2p_2 = prop(src2, dst2, y2.reshape(3 * N, L))

    t3 = pl.pallas_call(
        _t3_body,
        grid=grid,
        in_specs=[_pair_spec(R), _pair_spec(R), _row_spec(R, DH),
                  _row_spec(R, 1), _full_spec((1, DH)), _full_spec((DH, DOUT)),
                  _full_spec((1, DOUT))],
        out_specs=_row_spec(R, DOUT),
        out_shape=jax.ShapeDtypeStruct((N, DOUT), jnp.float32),
    )
    return t3(s01_2, s2p_2, y2, dinv, b2.reshape(1, DH), Wout,
              bout.reshape(1, DOUT))


# speedup vs baseline: 28.4313x; 1.0110x over previous
import functools

import jax
import jax.numpy as jnp
from jax import lax
from jax.experimental import pallas as pl
from jax.experimental.pallas import tpu as pltpu
from jax.experimental.pallas import tpu_sc as plsc

NC = 2
NS = 16
L = 16
B = 128
CB = 16
CH = CB * B
ND = 7
NB = 8
_SC_PARAMS = pltpu.CompilerParams(use_tc_tiling_on_sc=False)


def _cdiv(a, b):
    return (a + b - 1) // b


def _make_deg(N, EP):
    ept = EP // (NC * NS)
    nchunk = ept // CH
    DS = 8 * _cdiv(_cdiv(N + 16, NS), 8)
    DACC = NS * DS
    ZB = 2048
    mesh = plsc.VectorSubcoreMesh(
        core_axis_name="c", subcore_axis_name="s", num_cores=NC, num_subcores=NS)

    @functools.partial(
        pl.kernel,
        out_type=jax.ShapeDtypeStruct((NC, DACC), jnp.float32),
        mesh=mesh,
        compiler_params=_SC_PARAMS,
        scratch_types=[
            pltpu.VMEM_SHARED((DACC,), jnp.float32),
            pltpu.VMEM((ZB,), jnp.float32),
            pltpu.VMEM((CB, B), jnp.int32),
            pltpu.VMEM((B,), jnp.float32),
        ],
    )
    def deg_kernel(dst2, degp, dacc, zb, dstg, ones):
        c = lax.axis_index("c")
        t = lax.axis_index("s")

        def fill(i, _):
            zb[pl.ds(i * L, L)] = jnp.zeros((L,), jnp.float32)
            return 0
        lax.fori_loop(0, ZB // L, fill, 0)
        for v in range(B // L):
            ones[pl.ds(v * L, L)] = jnp.full((L,), 1.0, jnp.float32)

        zlo = t * DS
        nfull, rem = DS // ZB, DS % ZB
        for k in range(nfull):
            pltpu.sync_copy(zb, dacc.at[pl.ds(zlo + k * ZB, ZB)])
        if rem:
            pltpu.sync_copy(zb.at[pl.ds(0, rem)],
                            dacc.at[pl.ds(zlo + nfull * ZB, rem)])
        plsc.subcore_barrier()

        rows_per_tile = ept // B
        rbase0 = (c * NS + t) * rows_per_tile

        def chunk(ci, _):
            pltpu.sync_copy(dst2.at[pl.ds(rbase0 + ci * CB, CB)], dstg)
            for j in range(CB):
                pltpu.sync_copy(ones, dacc.at[dstg.at[j]], add=True)
            return 0
        lax.fori_loop(0, nchunk, chunk, 0)
        plsc.subcore_barrier()
        pltpu.sync_copy(dacc.at[pl.ds(t * DS, DS)],
                        degp.at[c, pl.ds(t * DS, DS)])

    return deg_kernel, DACC


def _make_prop(N, EP):
    ea_t = EP // NS
    eb_t = EP // (NS * NC)
    ncA, ncB = ea_t // CH, eb_t // CH
    FS = N // NS
    AccR = NS * _cdiv(N + 16, NS)
    ZS = AccR // NS
    ZB = 256
    mesh = plsc.VectorSubcoreMesh(
        core_axis_name="c", subcore_axis_name="s", num_cores=NC, num_subcores=NS)

    @functools.partial(
        pl.kernel,
        out_type=(jax.ShapeDtypeStruct((NC, N, L), jnp.float32),
                  jax.ShapeDtypeStruct((NC, N, L), jnp.float32)),
        mesh=mesh,
        compiler_params=_SC_PARAMS,
        scratch_types=[
            pltpu.VMEM_SHARED((AccR, L), jnp.float32),
            pltpu.VMEM((ZB, L), jnp.float32),
            pltpu.VMEM((CB, B), jnp.int32),
            pltpu.VMEM((CB, B), jnp.int32),
        ] + [pltpu.VMEM((B,), jnp.int32) for _ in range(NB)]
          + [pltpu.VMEM((B, L), jnp.float32) for _ in range(NB)]
          + [pltpu.SemaphoreType.DMA for _ in range(2 * NB)],
    )
    def prop_kernel(src2, dst2, yflat, s01, s2p,
                    acc, zb, sstg, dstg, *bufs):
        c = lax.axis_index("c")
        t = lax.axis_index("s")
        ibs, rbs = bufs[:NB], bufs[NB:2 * NB]
        gsems, ssems = bufs[2 * NB:3 * NB], bufs[3 * NB:]

        def fill(i, _):
            zb[i, :] = jnp.zeros((L,), jnp.float32)
            return 0
        lax.fori_loop(0, ZB, fill, 0)

        def prep(j, p, soff):
            for v in range(B // L):
                ibs[p][pl.ds(v * L, L)] = sstg[j, pl.ds(v * L, L)] * 3 + soff

        def do_pass(nchunks, rows_base, off, out_ref):
            zlo = t * ZS
            nfull, rem = ZS // ZB, ZS % ZB
            for k in range(nfull):
                pltpu.sync_copy(zb, acc.at[pl.ds(zlo + k * ZB, ZB)])
            if rem:
                pltpu.sync_copy(zb.at[pl.ds(0, rem)],
                                acc.at[pl.ds(zlo + nfull * ZB, rem)])
            plsc.subcore_barrier()

            def chunk(ci, _):
                rb = rows_base + ci * CB
                pltpu.sync_copy(src2.at[pl.ds(rb, CB)], sstg)
                pltpu.sync_copy(dst2.at[pl.ds(rb, CB)], dstg)
                gd, sd = {}, {}
                for j in range(min(ND, CB)):
                    q = j % NB
                    prep(j, q, off)
                    gd[j] = pltpu.async_copy(yflat.at[ibs[q]], rbs[q], gsems[q])
                for j in range(CB):
                    p = j % NB
                    gd.pop(j).wait()
                    sd[j] = pltpu.async_copy(
                        rbs[p], acc.at[dstg.at[j]], ssems[p], add=True)
                    nj = j + ND
                    if nj < CB:
                        q = nj % NB
                        if nj - NB in sd:
                            sd.pop(nj - NB).wait()
                        prep(nj, q, off)
                        gd[nj] = pltpu.async_copy(
                            yflat.at[ibs[q]], rbs[q], gsems[q])
                for j in sorted(sd):
                    sd.pop(j).wait()
                return 0
            lax.fori_loop(0, nchunks, chunk, 0)
            plsc.subcore_barrier()
            flo = t * FS
            pltpu.sync_copy(acc.at[pl.ds(flo, FS)],
                            out_ref.at[c, pl.ds(flo, FS)])
            plsc.subcore_barrier()

        do_pass(ncA, t * (ea_t // B), c, s01)
        do_pass(ncB, c * (EP // (2 * B)) + t * (eb_t // B), 2, s2p)

    return prop_kernel


def _t1_body(x_ref, w1_ref, da_ref, db_ref, y_ref, dinv_ref):
    deg = da_ref[...] + db_ref[...] + 1.0
    dinv = lax.rsqrt(deg)
    xw = jnp.dot(x_ref[...], w1_ref[...], preferred_element_type=jnp.float32)
    y_ref[...] = xw * dinv
    dinv_ref[...] = dinv


def _t2_body(s01_ref, s2p_ref, y_ref, dinv_ref, b1_ref, w2_ref, yo_ref):
    S = jnp.concatenate(
        [s01_ref[0], s01_ref[1], s2p_ref[0] + s2p_ref[1]], axis=1)
    dinv = dinv_ref[...]
    h = jnp.maximum(dinv * (S + y_ref[...]) + b1_ref[...], 0.0)
    yo_ref[...] = jnp.dot(h, w2_ref[...],
                          preferred_element_type=jnp.float32) * dinv


def _t3_body(s01_ref, s2p_ref, y_ref, dinv_ref, b2_ref, wo_ref, bo_ref,
             out_ref):
    S = jnp.concatenate(
        [s01_ref[0], s01_ref[1], s2p_ref[0] + s2p_ref[1]], axis=1)
    dinv = dinv_ref[...]
    h = jnp.maximum(dinv * (S + y_ref[...]) + b2_ref[...], 0.0)
    out_ref[...] = (jnp.dot(h, wo_ref[...], preferred_element_type=jnp.float32)
                    + bo_ref[...])


def _row_spec(R, width):
    return pl.BlockSpec((R, width), lambda i: (i, 0))


def _full_spec(shape):
    return pl.BlockSpec(shape, lambda i: tuple(0 for _ in shape))


def _slab_spec(R):
    return pl.BlockSpec((3, R, L), lambda i: (0, i, 0))


def _pair_spec(R):
    return pl.BlockSpec((2, R, L), lambda i: (0, i, 0))


def kernel(x, edge_index, W1, b1, W2, b2, Wout, bout):
    N, DIN = x.shape
    DH = W1.shape[1]
    DOUT = Wout.shape[1]
    E = edge_index.shape[1]
    assert DH == 3 * L and N % NS == 0

    EP = _cdiv(E, NC * NS * CH) * NC * NS * CH
    src = edge_index[0].astype(jnp.int32)
    dst = edge_index[1].astype(jnp.int32)
    pad = EP - E
    if pad:
        padi = jnp.arange(pad, dtype=jnp.int32)
        src = jnp.concatenate([src, padi % 64])
        dst = jnp.concatenate([dst, N + (padi % 16)])
    src2 = src.reshape(EP // B, B)
    dst2 = dst.reshape(EP // B, B)

    deg_kernel, DACC = _make_deg(N, EP)
    degp = deg_kernel(dst2)
    degA = degp[0, :N].reshape(N, 1)
    degB = degp[1, :N].reshape(N, 1)

    R = 2000
    grid = (N // R,)
    t1 = pl.pallas_call(
        _t1_body,
        grid=grid,
        in_specs=[_row_spec(R, DIN), _full_spec((DIN, DH)),
                  _row_spec(R, 1), _row_spec(R, 1)],
        out_specs=[_row_spec(R, DH), _row_spec(R, 1)],
        out_shape=[jax.ShapeDtypeStruct((N, DH), jnp.float32),
                   jax.ShapeDtypeStruct((N, 1), jnp.float32)],
    )
    y1, dinv = t1(x, W1, degA, degB)

    prop = _make_prop(N, EP)
    s01_1, s2p_1 = prop(src2, dst2, y1.reshape(3 * N, L))

    t2 = pl.pallas_call(
        _t2_body,
        grid=grid,
        in_specs=[_pair_spec(R), _pair_spec(R), _row_spec(R, DH),
                  _row_spec(R, 1), _full_spec((1, DH)), _full_spec((DH, DH))],
        out_specs=_row_spec(R, DH),
        out_shape=jax.ShapeDtypeStruct((N, DH), jnp.float32),
    )
    y2 = t2(s01_1, s2p_1, y1, dinv, b1.reshape(1, DH), W2)

    s01_2, s2p_2 = prop(src2, dst2, y2.reshape(3 * N, L))

    t3 = pl.pallas_call(
        _t3_body,
        grid=grid,
        in_specs=[_pair_spec(R), _pair_spec(R), _row_spec(R, DH),
                  _row_spec(R, 1), _full_spec((1, DH)), _full_spec((DH, DOUT)),
                  _full_spec((1, DOUT))],
        out_specs=_row_spec(R, DOUT),
        out_shape=jax.ShapeDtypeStruct((N, DOUT), jnp.float32),
    )
    return t3(s01_2, s2p_2, y2, dinv, b2.reshape(1, DH), Wout,
              bout.reshape(1, DOUT))


# speedup vs baseline: 29.1031x; 1.0236x over previous
import functools

import jax
import jax.numpy as jnp
from jax import lax
from jax.experimental import pallas as pl
from jax.experimental.pallas import tpu as pltpu
from jax.experimental.pallas import tpu_sc as plsc

NC = 2
NS = 16
L = 16
B = 128
CB = 16
CH = CB * B
ND = 8
NB = 10
_SC_PARAMS = pltpu.CompilerParams(use_tc_tiling_on_sc=False)


def _cdiv(a, b):
    return (a + b - 1) // b


def _make_deg(N, EP):
    ept = EP // (NC * NS)
    nchunk = ept // CH
    DS = 8 * _cdiv(_cdiv(N + 16, NS), 8)
    DACC = NS * DS
    ZB = 2048
    mesh = plsc.VectorSubcoreMesh(
        core_axis_name="c", subcore_axis_name="s", num_cores=NC, num_subcores=NS)

    @functools.partial(
        pl.kernel,
        out_type=jax.ShapeDtypeStruct((NC, DACC), jnp.float32),
        mesh=mesh,
        compiler_params=_SC_PARAMS,
        scratch_types=[
            pltpu.VMEM_SHARED((DACC,), jnp.float32),
            pltpu.VMEM((ZB,), jnp.float32),
            pltpu.VMEM((CB, B), jnp.int32),
            pltpu.VMEM((B,), jnp.float32),
        ],
    )
    def deg_kernel(dst2, degp, dacc, zb, dstg, ones):
        c = lax.axis_index("c")
        t = lax.axis_index("s")

        def fill(i, _):
            zb[pl.ds(i * L, L)] = jnp.zeros((L,), jnp.float32)
            return 0
        lax.fori_loop(0, ZB // L, fill, 0)
        for v in range(B // L):
            ones[pl.ds(v * L, L)] = jnp.full((L,), 1.0, jnp.float32)

        zlo = t * DS
        nfull, rem = DS // ZB, DS % ZB
        for k in range(nfull):
            pltpu.sync_copy(zb, dacc.at[pl.ds(zlo + k * ZB, ZB)])
        if rem:
            pltpu.sync_copy(zb.at[pl.ds(0, rem)],
                            dacc.at[pl.ds(zlo + nfull * ZB, rem)])
        plsc.subcore_barrier()

        rows_per_tile = ept // B
        rbase0 = (c * NS + t) * rows_per_tile

        def chunk(ci, _):
            pltpu.sync_copy(dst2.at[pl.ds(rbase0 + ci * CB, CB)], dstg)
            for j in range(CB):
                pltpu.sync_copy(ones, dacc.at[dstg.at[j]], add=True)
            return 0
        lax.fori_loop(0, nchunk, chunk, 0)
        plsc.subcore_barrier()
        pltpu.sync_copy(dacc.at[pl.ds(t * DS, DS)],
                        degp.at[c, pl.ds(t * DS, DS)])

    return deg_kernel, DACC


def _make_prop(N, EP):
    ea_t = EP // NS
    eb_t = EP // (NS * NC)
    ncA, ncB = ea_t // CH, eb_t // CH
    FS = N // NS
    AccR = NS * _cdiv(N + 16, NS)
    ZS = AccR // NS
    ZB = 128
    mesh = plsc.VectorSubcoreMesh(
        core_axis_name="c", subcore_axis_name="s", num_cores=NC, num_subcores=NS)

    @functools.partial(
        pl.kernel,
        out_type=(jax.ShapeDtypeStruct((NC, N, L), jnp.float32),
                  jax.ShapeDtypeStruct((NC, N, L), jnp.float32)),
        mesh=mesh,
        compiler_params=_SC_PARAMS,
        scratch_types=[
            pltpu.VMEM_SHARED((AccR, L), jnp.float32),
            pltpu.VMEM((ZB, L), jnp.float32),
            pltpu.VMEM((CB, B), jnp.int32),
            pltpu.VMEM((CB, B), jnp.int32),
        ] + [pltpu.VMEM((B,), jnp.int32) for _ in range(NB)]
          + [pltpu.VMEM((B, L), jnp.float32) for _ in range(NB)]
          + [pltpu.SemaphoreType.DMA for _ in range(2 * NB)],
    )
    def prop_kernel(src2, dst2, yflat, s01, s2p,
                    acc, zb, sstg, dstg, *bufs):
        c = lax.axis_index("c")
        t = lax.axis_index("s")
        ibs, rbs = bufs[:NB], bufs[NB:2 * NB]
        gsems, ssems = bufs[2 * NB:3 * NB], bufs[3 * NB:]

        def fill(i, _):
            zb[i, :] = jnp.zeros((L,), jnp.float32)
            return 0
        lax.fori_loop(0, ZB, fill, 0)

        def prep(j, p, soff):
            for v in range(B // L):
                ibs[p][pl.ds(v * L, L)] = sstg[j, pl.ds(v * L, L)] * 3 + soff

        def do_pass(nchunks, rows_base, off, out_ref):
            zlo = t * ZS
            nfull, rem = ZS // ZB, ZS % ZB
            for k in range(nfull):
                pltpu.sync_copy(zb, acc.at[pl.ds(zlo + k * ZB, ZB)])
            if rem:
                pltpu.sync_copy(zb.at[pl.ds(0, rem)],
                                acc.at[pl.ds(zlo + nfull * ZB, rem)])
            plsc.subcore_barrier()

            def chunk(ci, _):
                rb = rows_base + ci * CB
                pltpu.sync_copy(src2.at[pl.ds(rb, CB)], sstg)
                pltpu.sync_copy(dst2.at[pl.ds(rb, CB)], dstg)
                gd, sd = {}, {}
                for j in range(min(ND, CB)):
                    q = j % NB
                    prep(j, q, off)
                    gd[j] = pltpu.async_copy(yflat.at[ibs[q]], rbs[q], gsems[q])
                for j in range(CB):
                    p = j % NB
                    gd.pop(j).wait()
                    sd[j] = pltpu.async_copy(
                        rbs[p], acc.at[dstg.at[j]], ssems[p], add=True)
                    nj = j + ND
                    if nj < CB:
                        q = nj % NB
                        if nj - NB in sd:
                            sd.pop(nj - NB).wait()
                        prep(nj, q, off)
                        gd[nj] = pltpu.async_copy(
                            yflat.at[ibs[q]], rbs[q], gsems[q])
                for j in sorted(sd):
                    sd.pop(j).wait()
                return 0
            lax.fori_loop(0, nchunks, chunk, 0)
            plsc.subcore_barrier()
            flo = t * FS
            pltpu.sync_copy(acc.at[pl.ds(flo, FS)],
                            out_ref.at[c, pl.ds(flo, FS)])
            plsc.subcore_barrier()

        do_pass(ncA, t * (ea_t // B), c, s01)
        do_pass(ncB, c * (EP // (2 * B)) + t * (eb_t // B), 2, s2p)

    return prop_kernel


def _t1_body(x_ref, w1_ref, da_ref, db_ref, y_ref, dinv_ref):
    deg = da_ref[...] + db_ref[...] + 1.0
    dinv = lax.rsqrt(deg)
    xw = jnp.dot(x_ref[...], w1_ref[...], preferred_element_type=jnp.float32)
    y_ref[...] = xw * dinv
    dinv_ref[...] = dinv


def _t2_body(s01_ref, s2p_ref, y_ref, dinv_ref, b1_ref, w2_ref, yo_ref):
    S = jnp.concatenate(
        [s01_ref[0], s01_ref[1], s2p_ref[0] + s2p_ref[1]], axis=1)
    dinv = dinv_ref[...]
    h = jnp.maximum(dinv * (S + y_ref[...]) + b1_ref[...], 0.0)
    yo_ref[...] = jnp.dot(h, w2_ref[...],
                          preferred_element_type=jnp.float32) * dinv


def _t3_body(s01_ref, s2p_ref, y_ref, dinv_ref, b2_ref, wo_ref, bo_ref,
             out_ref):
    S = jnp.concatenate(
        [s01_ref[0], s01_ref[1], s2p_ref[0] + s2p_ref[1]], axis=1)
    dinv = dinv_ref[...]
    h = jnp.maximum(dinv * (S + y_ref[...]) + b2_ref[...], 0.0)
    out_ref[...] = (jnp.dot(h, wo_ref[...], preferred_element_type=jnp.float32)
                    + bo_ref[...])


def _row_spec(R, width):
    return pl.BlockSpec((R, width), lambda i: (i, 0))


def _full_spec(shape):
    return pl.BlockSpec(shape, lambda i: tuple(0 for _ in shape))


def _slab_spec(R):
    return pl.BlockSpec((3, R, L), lambda i: (0, i, 0))


def _pair_spec(R):
    return pl.BlockSpec((2, R, L), lambda i: (0, i, 0))


def kernel(x, edge_index, W1, b1, W2, b2, Wout, bout):
    N, DIN = x.shape
    DH = W1.shape[1]
    DOUT = Wout.shape[1]
    E = edge_index.shape[1]
    assert DH == 3 * L and N % NS == 0

    EP = _cdiv(E, NC * NS * CH) * NC * NS * CH
    src = edge_index[0].astype(jnp.int32)
    dst = edge_index[1].astype(jnp.int32)
    pad = EP - E
    if pad:
        padi = jnp.arange(pad, dtype=jnp.int32)
        src = jnp.concatenate([src, padi % 64])
        dst = jnp.concatenate([dst, N + (padi % 16)])
    src2 = src.reshape(EP // B, B)
    dst2 = dst.reshape(EP // B, B)

    deg_kernel, DACC = _make_deg(N, EP)
    degp = deg_kernel(dst2)
    degA = degp[0, :N].reshape(N, 1)
    degB = degp[1, :N].reshape(N, 1)

    R = 2000
    grid = (N // R,)
    t1 = pl.pallas_call(
        _t1_body,
        grid=grid,
        in_specs=[_row_spec(R, DIN), _full_spec((DIN, DH)),
                  _row_spec(R, 1), _row_spec(R, 1)],
        out_specs=[_row_spec(R, DH), _row_spec(R, 1)],
        out_shape=[jax.ShapeDtypeStruct((N, DH), jnp.float32),
                   jax.ShapeDtypeStruct((N, 1), jnp.float32)],
    )
    y1, dinv = t1(x, W1, degA, degB)

    prop = _make_prop(N, EP)
    s01_1, s2p_1 = prop(src2, dst2, y1.reshape(3 * N, L))

    t2 = pl.pallas_call(
        _t2_body,
        grid=grid,
        in_specs=[_pair_spec(R), _pair_spec(R), _row_spec(R, DH),
                  _row_spec(R, 1), _full_spec((1, DH)), _full_spec((DH, DH))],
        out_specs=_row_spec(R, DH),
        out_shape=jax.ShapeDtypeStruct((N, DH), jnp.float32),
    )
    y2 = t2(s01_1, s2p_1, y1, dinv, b1.reshape(1, DH), W2)

    s01_2, s2p_2 = prop(src2, dst2, y2.reshape(3 * N, L))

    t3 = pl.pallas_call(
        _t3_body,
        grid=grid,
        in_specs=[_pair_spec(R), _pair_spec(R), _row_spec(R, DH),
                  _row_spec(R, 1), _full_spec((1, DH)), _full_spec((DH, DOUT)),
                  _full_spec((1, DOUT))],
        out_specs=_row_spec(R, DOUT),
        out_shape=jax.ShapeDtypeStruct((N, DOUT), jnp.float32),
    )
    return t3(s01_2, s2p_2, y2, dinv, b2.reshape(1, DH), Wout,
              bout.reshape(1, DOUT))
